# baseline (speedup 1.0000x reference)
import functools

import jax
import jax.numpy as jnp
from jax import lax
from jax.experimental import pallas as pl
from jax.experimental.pallas import tpu as pltpu, tpu_sc as plsc

B = 16384
N = 1000
NC = 2
NS = 16
L = 16
NW = NC * NS
COLS_PER_W = B // NW
NGRP = COLS_PER_W // L
RH = 40
NCHUNK = N // RH
NPAIR = (NCHUNK - 3) // 2


def _onehot_t_body(x_hbm, out_hbm, idx_v, buf0, buf1, sem0, sem1, semx):
    wid = lax.axis_index("s") * NC + lax.axis_index("c")
    col0 = wid * COLS_PER_W

    xcopy = pltpu.async_copy(x_hbm.at[pl.ds(col0, COLS_PER_W)], idx_v, semx)

    zeros16 = jnp.zeros((L,), jnp.float32)
    ones16 = jnp.full((L,), 1.0, jnp.float32)
    lanes = lax.iota(jnp.int32, L)

    def zero_buf(buf):
        def zero_row(r, carry):
            for g in range(NGRP):
                buf[r, pl.ds(g * L, L)] = zeros16
            return carry

        lax.fori_loop(0, RH, zero_row, 0)

    def chunk_pass(buf, r0_new, r0_old):
        for g in range(NGRP):
            xv = idx_v[pl.ds(g * L, L)]
            cols = lanes + g * L
            if r0_old is not None:
                m_old = (xv >= r0_old) & (xv < r0_old + RH)
                plsc.store_scatter(buf, [xv - r0_old, cols], zeros16, mask=m_old)
            m_new = (xv >= r0_new) & (xv < r0_new + RH)
            plsc.store_scatter(buf, [xv - r0_new, cols], ones16, mask=m_new)

    def dma(buf, r0, sem):
        return pltpu.async_copy(
            buf, out_hbm.at[pl.ds(r0, RH), pl.ds(col0, COLS_PER_W)], sem
        )

    def wait(buf, r0, sem):
        pltpu.make_async_copy(
            buf, out_hbm.at[pl.ds(r0, RH), pl.ds(col0, COLS_PER_W)], sem
        ).wait()

    zero_buf(buf0)
    xcopy.wait()
    chunk_pass(buf0, 0, None)
    dma(buf0, 0, sem0)
    zero_buf(buf1)
    chunk_pass(buf1, RH, None)
    dma(buf1, RH, sem1)

    def pair(t, carry):
        c = 2 * t + 2
        r0a = c * RH
        wait(buf0, r0a, sem0)
        chunk_pass(buf0, r0a, r0a - 2 * RH)
        dma(buf0, r0a, sem0)
        r0b = (c + 1) * RH
        wait(buf1, r0b, sem1)
        chunk_pass(buf1, r0b, r0b - 2 * RH)
        dma(buf1, r0b, sem1)
        return carry

    lax.fori_loop(0, NPAIR, pair, 0, unroll=False)

    tail = NCHUNK - 1
    wait(buf0, tail * RH, sem0)
    chunk_pass(buf0, tail * RH, (tail - 2) * RH)
    dma(buf0, tail * RH, sem0)

    wait(buf1, (tail - 1) * RH, sem1)
    wait(buf0, tail * RH, sem0)


@functools.partial(jax.jit, static_argnames=())
def _onehot(x):
    mesh = plsc.VectorSubcoreMesh(
        core_axis_name="c", subcore_axis_name="s", num_cores=NC, num_subcores=NS
    )
    out_t = pl.kernel(
        _onehot_t_body,
        out_type=jax.ShapeDtypeStruct((N, B), jnp.float32),
        mesh=mesh,
        scratch_types=[
            pltpu.VMEM((COLS_PER_W,), jnp.int32),
            pltpu.VMEM((RH, COLS_PER_W), jnp.float32),
            pltpu.VMEM((RH, COLS_PER_W), jnp.float32),
            pltpu.SemaphoreType.DMA,
            pltpu.SemaphoreType.DMA,
            pltpu.SemaphoreType.DMA,
        ],
        compiler_params=pltpu.CompilerParams(
            needs_layout_passes=False, skip_device_barrier=True
        ),
        name="onehot_sc_t",
    )(x)
    return out_t.T


def kernel(x):
    return _onehot(x.astype(jnp.int32))


# speedup vs baseline: 1.0007x; 1.0007x over previous
import functools

import jax
import jax.numpy as jnp
from jax import lax
from jax.experimental import pallas as pl
from jax.experimental.pallas import tpu as pltpu, tpu_sc as plsc

B = 16384
N = 1000
NC = 2
NS = 16
L = 16
NW = NC * NS
COLS_PER_W = B // NW
NGRP = COLS_PER_W // L
RH = 40
NCHUNK = N // RH
NPAIR = (NCHUNK - 3) // 2


def _onehot_t_body(x_hbm, out_hbm, idx_v, buf0, buf1, sem0, sem1, semx):
    wid = lax.axis_index("s") * NC + lax.axis_index("c")
    col0 = wid * COLS_PER_W

    xcopy = pltpu.async_copy(x_hbm.at[pl.ds(col0, COLS_PER_W)], idx_v, semx)

    zeros16 = jnp.zeros((L,), jnp.float32)
    ones16 = jnp.full((L,), 1.0, jnp.float32)
    lanes = lax.iota(jnp.int32, L)

    def zero_buf(buf):
        def zero_row(r, carry):
            for g in range(NGRP):
                buf[r, pl.ds(g * L, L)] = zeros16
            return carry

        lax.fori_loop(0, RH, zero_row, 0)

    def chunk_pass(buf, r0_new, r0_old):
        for g in range(NGRP):
            xv = idx_v[pl.ds(g * L, L)]
            cols = lanes + g * L
            if r0_old is not None:
                m_old = (xv >= r0_old) & (xv < r0_old + RH)
                plsc.store_scatter(buf, [xv - r0_old, cols], zeros16, mask=m_old)
            m_new = (xv >= r0_new) & (xv < r0_new + RH)
            plsc.store_scatter(buf, [xv - r0_new, cols], ones16, mask=m_new)

    def dma(buf, r0, sem):
        return pltpu.async_copy(
            buf, out_hbm.at[pl.ds(r0, RH), pl.ds(col0, COLS_PER_W)], sem
        )

    def wait(buf, r0, sem):
        pltpu.make_async_copy(
            buf, out_hbm.at[pl.ds(r0, RH), pl.ds(col0, COLS_PER_W)], sem
        ).wait()

    zero_buf(buf0)
    xcopy.wait()
    chunk_pass(buf0, 0, None)
    dma(buf0, 0, sem0)
    zero_buf(buf1)
    chunk_pass(buf1, RH, None)
    dma(buf1, RH, sem1)

    def pair(t, carry):
        c = 2 * t + 2
        r0a = c * RH
        wait(buf0, r0a, sem0)
        chunk_pass(buf0, r0a, r0a - 2 * RH)
        dma(buf0, r0a, sem0)
        r0b = (c + 1) * RH
        wait(buf1, r0b, sem1)
        chunk_pass(buf1, r0b, r0b - 2 * RH)
        dma(buf1, r0b, sem1)
        return carry

    lax.fori_loop(0, NPAIR, pair, 0, unroll=False)

    tail = NCHUNK - 1
    wait(buf0, tail * RH, sem0)
    chunk_pass(buf0, tail * RH, (tail - 2) * RH)
    dma(buf0, tail * RH, sem0)

    wait(buf1, (tail - 1) * RH, sem1)
    wait(buf0, tail * RH, sem0)


@functools.partial(jax.jit, static_argnames=())
def _onehot(x):
    mesh = plsc.VectorSubcoreMesh(
        core_axis_name="c", subcore_axis_name="s", num_cores=NC, num_subcores=NS
    )
    out_t = pl.kernel(
        _onehot_t_body,
        out_type=jax.ShapeDtypeStruct((N, B), jnp.float32),
        mesh=mesh,
        scratch_types=[
            pltpu.VMEM((COLS_PER_W,), jnp.int32),
            pltpu.VMEM((RH, COLS_PER_W), jnp.float32),
            pltpu.VMEM((RH, COLS_PER_W), jnp.float32),
            pltpu.SemaphoreType.DMA,
            pltpu.SemaphoreType.DMA,
            pltpu.SemaphoreType.DMA,
        ],
        compiler_params=pltpu.CompilerParams(needs_layout_passes=False),
        name="onehot_sc_t",
    )(x)
    return out_t.T


def kernel(x):
    return _onehot(x.astype(jnp.int32))


# speedup vs baseline: 1.0572x; 1.0564x over previous
import functools

import jax
import jax.numpy as jnp
from jax import lax
from jax.experimental import pallas as pl
from jax.experimental.pallas import tpu as pltpu, tpu_sc as plsc

B = 16384
N = 1000
NC = 2
NS = 16
L = 16
NW = NC * NS
COLS_PER_W = B // NW
NGRP = COLS_PER_W // L
RH = 40
NCHUNK = N // RH
NPAIR = (NCHUNK - 3) // 2


def _onehot_t_body(x_hbm, out_hbm, idx_v, buf0, buf1, sem0, sem1, semx):
    wid = lax.axis_index("s") * NC + lax.axis_index("c")
    col0 = wid * COLS_PER_W

    xcopy = pltpu.async_copy(x_hbm.at[pl.ds(col0, COLS_PER_W)], idx_v, semx)

    zeros16 = jnp.zeros((L,), jnp.float32)
    ones16 = jnp.full((L,), 1.0, jnp.float32)
    lanes = lax.iota(jnp.int32, L)

    def zero_buf(buf):
        def zero_row(r, carry):
            for g in range(NGRP):
                buf[r, pl.ds(g * L, L)] = zeros16
            return carry

        lax.fori_loop(0, RH, zero_row, 0)

    def chunk_pass(buf, r0_new, r0_old):
        def group(g, carry):
            xv = idx_v[pl.ds(g * L, L)]
            cols = lanes + g * L
            if r0_old is not None:
                m_old = (xv >= r0_old) & (xv < r0_old + RH)
                plsc.store_scatter(buf, [xv - r0_old, cols], zeros16, mask=m_old)
            m_new = (xv >= r0_new) & (xv < r0_new + RH)
            plsc.store_scatter(buf, [xv - r0_new, cols], ones16, mask=m_new)
            return carry

        lax.fori_loop(0, NGRP, group, 0)

    def dma(buf, r0, sem):
        return pltpu.async_copy(
            buf, out_hbm.at[pl.ds(r0, RH), pl.ds(col0, COLS_PER_W)], sem
        )

    def wait(buf, r0, sem):
        pltpu.make_async_copy(
            buf, out_hbm.at[pl.ds(r0, RH), pl.ds(col0, COLS_PER_W)], sem
        ).wait()

    zero_buf(buf0)
    xcopy.wait()
    chunk_pass(buf0, 0, None)
    dma(buf0, 0, sem0)
    zero_buf(buf1)
    chunk_pass(buf1, RH, None)
    dma(buf1, RH, sem1)

    def pair(t, carry):
        c = 2 * t + 2
        r0a = c * RH
        wait(buf0, r0a, sem0)
        chunk_pass(buf0, r0a, r0a - 2 * RH)
        dma(buf0, r0a, sem0)
        r0b = (c + 1) * RH
        wait(buf1, r0b, sem1)
        chunk_pass(buf1, r0b, r0b - 2 * RH)
        dma(buf1, r0b, sem1)
        return carry

    lax.fori_loop(0, NPAIR, pair, 0, unroll=False)

    tail = NCHUNK - 1
    wait(buf0, tail * RH, sem0)
    chunk_pass(buf0, tail * RH, (tail - 2) * RH)
    dma(buf0, tail * RH, sem0)

    wait(buf1, (tail - 1) * RH, sem1)
    wait(buf0, tail * RH, sem0)


@functools.partial(jax.jit, static_argnames=())
def _onehot(x):
    mesh = plsc.VectorSubcoreMesh(
        core_axis_name="c", subcore_axis_name="s", num_cores=NC, num_subcores=NS
    )
    out_t = pl.kernel(
        _onehot_t_body,
        out_type=jax.ShapeDtypeStruct((N, B), jnp.float32),
        mesh=mesh,
        scratch_types=[
            pltpu.VMEM((COLS_PER_W,), jnp.int32),
            pltpu.VMEM((RH, COLS_PER_W), jnp.float32),
            pltpu.VMEM((RH, COLS_PER_W), jnp.float32),
            pltpu.SemaphoreType.DMA,
            pltpu.SemaphoreType.DMA,
            pltpu.SemaphoreType.DMA,
        ],
        compiler_params=pltpu.CompilerParams(needs_layout_passes=False),
        name="onehot_sc_t",
    )(x)
    return out_t.T


def kernel(x):
    return _onehot(x.astype(jnp.int32))
